# baseline (speedup 1.0000x reference)
import functools

import jax
import jax.numpy as jnp
from jax import lax
from jax.experimental import pallas as pl
from jax.experimental.pallas import tpu as pltpu, tpu_sc as plsc

N = 10000
E = 320000
D = 128
H = 128
DE = 16
L = 4
G = 64

BN = 2000
BE = 8000
EC = 128
NW = 32
CPW = (E // EC + NW - 1) // NW
NCHUNK = CPW * NW
EPAD = NCHUNK * EC
RPS = 624
RCP = 104
RTAIL = N - 16 * RPS


def _init_body(x_ref, w_ref, b_ref, o_ref):
    o_ref[...] = jnp.dot(x_ref[...], w_ref[...],
                         preferred_element_type=jnp.float32) + b_ref[...]


def _node_init(x, w, b):
    return pl.pallas_call(
        _init_body,
        grid=(N // BN,),
        in_specs=[
            pl.BlockSpec((BN, D), lambda i: (i, 0)),
            pl.BlockSpec((D, H), lambda i: (0, 0)),
            pl.BlockSpec((1, H), lambda i: (0, 0)),
        ],
        out_specs=pl.BlockSpec((BN, H), lambda i: (i, 0)),
        out_shape=jax.ShapeDtypeStruct((N, H), jnp.float32),
    )(x, w, b.reshape(1, H))


def _edge_body(a1_ref, a2_ref, w1_ref, w2_ref, b_ref, o_ref):
    def emb(lo):
        sl = slice(16 * lo, 16 * lo + 16)
        return (jnp.dot(a1_ref[:, sl], w1_ref[...],
                        preferred_element_type=jnp.float32)
                + jnp.dot(a2_ref[:, sl], w2_ref[...],
                          preferred_element_type=jnp.float32)
                + b_ref[...])

    def pk(e):
        return lax.bitcast_convert_type(
            e.astype(jnp.bfloat16).astype(jnp.float32), jnp.uint32)

    o_ref[...] = (pk(emb(0)) >> 16) | (pk(emb(1)) & jnp.uint32(0xFFFF0000))


def _edge_embed(ea, ea2, w1, w2, b12):
    return pl.pallas_call(
        _edge_body,
        grid=(E // BE,),
        in_specs=[
            pl.BlockSpec((BE // 2, 2 * DE), lambda i: (i, 0)),
            pl.BlockSpec((BE // 2, 2 * DE), lambda i: (i, 0)),
            pl.BlockSpec((DE, H), lambda i: (0, 0)),
            pl.BlockSpec((DE, H), lambda i: (0, 0)),
            pl.BlockSpec((1, H), lambda i: (0, 0)),
        ],
        out_specs=pl.BlockSpec((BE // 2, H), lambda i: (i, 0)),
        out_shape=jax.ShapeDtypeStruct((EPAD // 2, H), jnp.uint32),
    )(ea.reshape(E // 2, 2 * DE), ea2.reshape(E // 2, 2 * DE),
      w1, w2, b12.reshape(1, H))


def _ln(h, g, b):
    mu = jnp.mean(h, axis=-1, keepdims=True)
    d = h - mu
    var = jnp.mean(d * d, axis=-1, keepdims=True)
    return d * jax.lax.rsqrt(var + 1e-5) * g + b


def _layer_body(agg_ref, hl_ref, bt_ref, w_ref, b_ref, g_ref, be_ref,
                h_ref, pool_ref):
    i = pl.program_id(0)
    hl = hl_ref[...]
    s = agg_ref[0] + agg_ref[1] + hl
    h = jnp.dot(s, w_ref[...], preferred_element_type=jnp.float32) + b_ref[...]
    h_ref[...] = _ln(h, g_ref[...], be_ref[...])
    onehot = (bt_ref[...] == lax.broadcasted_iota(jnp.int32, (1, G), 1)
              ).astype(jnp.float32)
    contrib = lax.dot_general(onehot, hl, (((0,), (0,)), ((), ())),
                              preferred_element_type=jnp.float32)

    @pl.when(i == 0)
    def _():
        pool_ref[...] = contrib

    @pl.when(i != 0)
    def _():
        pool_ref[...] += contrib


def _layer_update(agg, hl, batch2d, w, b, g, be):
    return pl.pallas_call(
        _layer_body,
        grid=(N // BN,),
        in_specs=[
            pl.BlockSpec((2, BN, H), lambda i: (0, i, 0)),
            pl.BlockSpec((BN, H), lambda i: (i, 0)),
            pl.BlockSpec((BN, 1), lambda i: (i, 0)),
            pl.BlockSpec((H, H), lambda i: (0, 0)),
            pl.BlockSpec((1, H), lambda i: (0, 0)),
            pl.BlockSpec((1, H), lambda i: (0, 0)),
            pl.BlockSpec((1, H), lambda i: (0, 0)),
        ],
        out_specs=[
            pl.BlockSpec((BN, H), lambda i: (i, 0)),
            pl.BlockSpec((G, H), lambda i: (0, 0)),
        ],
        out_shape=[
            jax.ShapeDtypeStruct((N, H), jnp.float32),
            jax.ShapeDtypeStruct((G, H), jnp.float32),
        ],
    )(agg, hl, batch2d, w, b.reshape(1, H), g.reshape(1, H),
      be.reshape(1, H))


def _vn_hl_body(pool_ref, vn_ref, w1_ref, b1_ref, w2_ref, b2_ref,
                h_ref, bt_ref, vno_ref, hlo_ref, vns_ref):
    i = pl.program_id(0)

    @pl.when(i == 0)
    def _():
        t = pool_ref[...] + vn_ref[...]
        t = jnp.maximum(jnp.dot(t, w1_ref[...],
                                preferred_element_type=jnp.float32)
                        + b1_ref[...], 0.0)
        t = jnp.maximum(jnp.dot(t, w2_ref[...],
                                preferred_element_type=jnp.float32)
                        + b2_ref[...], 0.0)
        vns_ref[...] = t
        vno_ref[...] = t

    onehot = (bt_ref[...] == lax.broadcasted_iota(jnp.int32, (1, G), 1)
              ).astype(jnp.float32)
    hlo_ref[...] = h_ref[...] + jnp.dot(onehot, vns_ref[...],
                                        preferred_element_type=jnp.float32)


def _vn_hl(pool, vn, w1, b1, w2, b2, h, batch2d):
    return pl.pallas_call(
        _vn_hl_body,
        grid=(N // BN,),
        in_specs=[
            pl.BlockSpec((G, H), lambda i: (0, 0)),
            pl.BlockSpec((G, H), lambda i: (0, 0)),
            pl.BlockSpec((H, H), lambda i: (0, 0)),
            pl.BlockSpec((1, H), lambda i: (0, 0)),
            pl.BlockSpec((H, H), lambda i: (0, 0)),
            pl.BlockSpec((1, H), lambda i: (0, 0)),
            pl.BlockSpec((BN, H), lambda i: (i, 0)),
            pl.BlockSpec((BN, 1), lambda i: (i, 0)),
        ],
        out_specs=[
            pl.BlockSpec((G, H), lambda i: (0, 0)),
            pl.BlockSpec((BN, H), lambda i: (i, 0)),
        ],
        out_shape=[
            jax.ShapeDtypeStruct((G, H), jnp.float32),
            jax.ShapeDtypeStruct((N, H), jnp.float32),
        ],
        scratch_shapes=[pltpu.VMEM((G, H), jnp.float32)],
    )(pool, vn, w1, b1.reshape(1, H), w2, b2.reshape(1, H), h, batch2d)


def _final_body(agg_ref, hl_ref, w_ref, b_ref, g_ref, be_ref,
                wo_ref, bo_ref, o_ref):
    s = agg_ref[0] + agg_ref[1] + hl_ref[...]
    h = jnp.dot(s, w_ref[...], preferred_element_type=jnp.float32) + b_ref[...]
    h = _ln(h, g_ref[...], be_ref[...])
    o_ref[...] = jnp.maximum(
        jnp.dot(h, wo_ref[...], preferred_element_type=jnp.float32)
        + bo_ref[...], 0.0)


def _final(agg, hl, w, b, g, be, wo, bo):
    return pl.pallas_call(
        _final_body,
        grid=(N // BN,),
        in_specs=[
            pl.BlockSpec((2, BN, H), lambda i: (0, i, 0)),
            pl.BlockSpec((BN, H), lambda i: (i, 0)),
            pl.BlockSpec((H, H), lambda i: (0, 0)),
            pl.BlockSpec((1, H), lambda i: (0, 0)),
            pl.BlockSpec((1, H), lambda i: (0, 0)),
            pl.BlockSpec((1, H), lambda i: (0, 0)),
            pl.BlockSpec((H, H), lambda i: (0, 0)),
            pl.BlockSpec((1, H), lambda i: (0, 0)),
        ],
        out_specs=pl.BlockSpec((BN, H), lambda i: (i, 0)),
        out_shape=jax.ShapeDtypeStruct((N, H), jnp.float32),
    )(agg, hl, w, b.reshape(1, H), g.reshape(1, H), be.reshape(1, H),
      wo, bo.reshape(1, H))


def _sc_edge_body(h_hbm, e_hbm, src_hbm, dst_hbm, out_hbm,
                  srcv, dstv, ev, rows, acc,
                  sg0, sg1, ssd0, ssd1, se0, se1):
    cid = lax.axis_index("c")
    sid = lax.axis_index("s")
    w = sid * 2 + cid
    sem_g = (sg0, sg1)
    sem_sd = (ssd0, ssd1)
    sem_e = (se0, se1)
    EH = EC // 2

    def issue_sd(j, b):
        g = j * NW + w
        pltpu.async_copy(src_hbm.at[g], srcv.at[b], sem_sd[b])
        pltpu.async_copy(dst_hbm.at[g], dstv.at[b], sem_sd[b])

    def wait_sd(j, b):
        g = j * NW + w
        pltpu.make_async_copy(src_hbm.at[g], srcv.at[b], sem_sd[b]).wait()
        pltpu.make_async_copy(dst_hbm.at[g], dstv.at[b], sem_sd[b]).wait()

    def issue_e(j, b):
        g = j * NW + w
        pltpu.async_copy(e_hbm.at[pl.ds(g * EH, EH)], ev.at[b], sem_e[b])

    def wait_e(j, b):
        g = j * NW + w
        pltpu.make_async_copy(e_hbm.at[pl.ds(g * EH, EH)], ev.at[b],
                              sem_e[b]).wait()

    def issue_gather(b):
        pltpu.async_copy(h_hbm.at[srcv.at[b, 0]], rows.at[b], sem_g[b])

    def wait_gather(b):
        pltpu.make_async_copy(h_hbm.at[srcv.at[b, 0]], rows.at[b],
                              sem_g[b]).wait()

    def compute(b):
        def _row(rr, cc):
            r0 = 2 * rr
            r1 = 2 * rr + 1
            for q in range(8):
                sq = pl.ds(16 * q, 16)
                wv = ev[b, rr, sq]
                lo = lax.bitcast_convert_type(wv << 16, jnp.float32)
                hi = lax.bitcast_convert_type(
                    wv & jnp.uint32(0xFFFF0000), jnp.float32)
                rows[b, r0, sq] = jnp.maximum(rows[b, r0, sq] + lo, 0.0)
                rows[b, r1, sq] = jnp.maximum(rows[b, r1, sq] + hi, 0.0)
            return cc

        lax.fori_loop(0, EH, _row, 0)

    def scatter(b):
        pltpu.sync_copy(rows.at[b], acc.at[dstv.at[b, 0]], add=True)

    def head(j, b):
        wait_e(j, b)
        wait_gather(b)

    def stage(j, b):
        wait_sd(j + 1, 1 - b)
        issue_gather(1 - b)

    def tail(j, b):
        compute(b)
        scatter(b)

    def _zrow(r, c):
        for jj in range(8):
            rows[0, r, pl.ds(jj * 16, 16)] = jnp.zeros((16,), jnp.float32)
        return c

    lax.fori_loop(0, EC, _zrow, 0)
    for t in range(RPS // RCP):
        pltpu.sync_copy(rows.at[0, pl.ds(0, RCP)],
                        acc.at[pl.ds(sid * RPS + t * RCP, RCP)])

    @pl.when(sid == 0)
    def _():
        pltpu.sync_copy(rows.at[0, pl.ds(0, RTAIL)],
                        acc.at[pl.ds(16 * RPS, RTAIL)])

    plsc.subcore_barrier()

    issue_sd(0, 0)
    issue_sd(1, 1)
    issue_e(0, 0)
    issue_e(1, 1)
    wait_sd(0, 0)
    issue_gather(0)

    def _pair(jo, c):
        for b in (0, 1):
            j = 2 * jo + b
            head(j, b)
            stage(j, b)
            tail(j, b)
            issue_sd(j + 2, b)
            issue_e(j + 2, b)
        return c

    lax.fori_loop(0, (CPW - 3) // 2, _pair, 0)
    for j in range(2 * ((CPW - 3) // 2), CPW):
        b = j & 1
        head(j, b)
        if j + 1 < CPW:
            stage(j, b)
        tail(j, b)
        if j + 2 < CPW:
            issue_sd(j + 2, b)
            issue_e(j + 2, b)
    plsc.subcore_barrier()

    for t in range(RPS // RCP):
        base = sid * RPS + t * RCP
        pltpu.sync_copy(acc.at[pl.ds(base, RCP)], rows.at[0, pl.ds(0, RCP)])
        pltpu.sync_copy(rows.at[0, pl.ds(0, RCP)],
                        out_hbm.at[cid, pl.ds(base, RCP)])

    @pl.when(sid == 0)
    def _():
        pltpu.sync_copy(acc.at[pl.ds(16 * RPS, RTAIL)],
                        rows.at[0, pl.ds(0, RTAIL)])
        pltpu.sync_copy(rows.at[0, pl.ds(0, RTAIL)],
                        out_hbm.at[cid, pl.ds(16 * RPS, RTAIL)])


@functools.cache
def _sc_edge_kernel():
    return pl.kernel(
        _sc_edge_body,
        out_type=jax.ShapeDtypeStruct((2, N, H), jnp.float32),
        mesh=plsc.VectorSubcoreMesh(core_axis_name="c",
                                    subcore_axis_name="s"),
        scratch_types=[
            pltpu.VMEM((2, 1, EC), jnp.int32),
            pltpu.VMEM((2, 1, EC), jnp.int32),
            pltpu.VMEM((2, EC // 2, H), jnp.uint32),
            pltpu.VMEM((2, EC, H), jnp.float32),
            pltpu.VMEM_SHARED((N + 16, H), jnp.float32),
            pltpu.SemaphoreType.DMA,
            pltpu.SemaphoreType.DMA,
            pltpu.SemaphoreType.DMA,
            pltpu.SemaphoreType.DMA,
            pltpu.SemaphoreType.DMA,
            pltpu.SemaphoreType.DMA,
        ],
    )


def _sc_edge(hl, e, src2d, dst2d):
    return _sc_edge_kernel()(hl, e, src2d, dst2d)


@jax.jit
def _run(x, edge_index, edge_attr, edge_attr_v2, batch, W_init, b_init,
         We1, be1, We2, be2, W_layers, b_layers, gamma, beta,
         Wvn1, bvn1, Wvn2, bvn2, W_out, b_out):
    npad = EPAD - E
    src2d = jnp.concatenate(
        [edge_index[0], jnp.zeros((npad,), jnp.int32)]).reshape(NCHUNK, 1, EC)
    dst2d = jnp.concatenate(
        [edge_index[1], jnp.full((npad,), N, jnp.int32)]).reshape(NCHUNK, 1, EC)
    batch2d = batch.reshape(N, 1)

    e = _edge_embed(edge_attr, edge_attr_v2, We1, We2, be1 + be2)
    hl = _node_init(x, W_init, b_init)
    vn = jnp.zeros((G, H), jnp.float32)

    for l in range(L - 1):
        agg = _sc_edge(hl, e, src2d, dst2d)
        h, pool = _layer_update(agg, hl, batch2d, W_layers[l], b_layers[l],
                                gamma[l], beta[l])
        vn, hl = _vn_hl(pool, vn, Wvn1[l], bvn1[l], Wvn2[l], bvn2[l],
                        h, batch2d)

    agg = _sc_edge(hl, e, src2d, dst2d)
    return _final(agg, hl, W_layers[L - 1], b_layers[L - 1],
                  gamma[L - 1], beta[L - 1], W_out, b_out)


def kernel(x, edge_index, edge_attr, edge_attr_v2, batch, W_init, b_init,
           We1, be1, We2, be2, W_layers, b_layers, gamma, beta,
           Wvn1, bvn1, Wvn2, bvn2, W_out, b_out):
    return _run(x, edge_index, edge_attr, edge_attr_v2, batch, W_init,
                b_init, We1, be1, We2, be2, W_layers, b_layers, gamma,
                beta, Wvn1, bvn1, Wvn2, bvn2, W_out, b_out)


# speedup vs baseline: 1.0004x; 1.0004x over previous
import functools

import jax
import jax.numpy as jnp
from jax import lax
from jax.experimental import pallas as pl
from jax.experimental.pallas import tpu as pltpu, tpu_sc as plsc

N = 10000
E = 320000
D = 128
H = 128
DE = 16
L = 4
G = 64

BN = 2000
BE = 8000
EC = 128
NW = 32
CPW = (E // EC + NW - 1) // NW
NCHUNK = CPW * NW
EPAD = NCHUNK * EC
RPS = 624
RCP = 104
RTAIL = N - 16 * RPS


def _init_body(x_ref, w_ref, b_ref, o_ref):
    o_ref[...] = jnp.dot(x_ref[...], w_ref[...],
                         preferred_element_type=jnp.float32) + b_ref[...]


def _node_init(x, w, b):
    return pl.pallas_call(
        _init_body,
        grid=(N // BN,),
        in_specs=[
            pl.BlockSpec((BN, D), lambda i: (i, 0)),
            pl.BlockSpec((D, H), lambda i: (0, 0)),
            pl.BlockSpec((1, H), lambda i: (0, 0)),
        ],
        out_specs=pl.BlockSpec((BN, H), lambda i: (i, 0)),
        out_shape=jax.ShapeDtypeStruct((N, H), jnp.float32),
    )(x, w, b.reshape(1, H))


def _edge_body(a1_ref, a2_ref, w1_ref, w2_ref, b_ref, o_ref):
    def emb(lo):
        sl = slice(16 * lo, 16 * lo + 16)
        return (jnp.dot(a1_ref[:, sl], w1_ref[...],
                        preferred_element_type=jnp.float32)
                + jnp.dot(a2_ref[:, sl], w2_ref[...],
                          preferred_element_type=jnp.float32)
                + b_ref[...])

    def pk(e):
        return lax.bitcast_convert_type(
            e.astype(jnp.bfloat16).astype(jnp.float32), jnp.uint32)

    o_ref[...] = (pk(emb(0)) >> 16) | (pk(emb(1)) & jnp.uint32(0xFFFF0000))


def _edge_embed(ea, ea2, w1, w2, b12):
    return pl.pallas_call(
        _edge_body,
        grid=(E // BE,),
        in_specs=[
            pl.BlockSpec((BE // 2, 2 * DE), lambda i: (i, 0)),
            pl.BlockSpec((BE // 2, 2 * DE), lambda i: (i, 0)),
            pl.BlockSpec((DE, H), lambda i: (0, 0)),
            pl.BlockSpec((DE, H), lambda i: (0, 0)),
            pl.BlockSpec((1, H), lambda i: (0, 0)),
        ],
        out_specs=pl.BlockSpec((BE // 2, H), lambda i: (i, 0)),
        out_shape=jax.ShapeDtypeStruct((EPAD // 2, H), jnp.uint32),
    )(ea.reshape(E // 2, 2 * DE), ea2.reshape(E // 2, 2 * DE),
      w1, w2, b12.reshape(1, H))


def _ln(h, g, b):
    mu = jnp.mean(h, axis=-1, keepdims=True)
    d = h - mu
    var = jnp.mean(d * d, axis=-1, keepdims=True)
    return d * jax.lax.rsqrt(var + 1e-5) * g + b


def _layer_body(agg_ref, hl_ref, bt_ref, w_ref, b_ref, g_ref, be_ref,
                h_ref, pool_ref):
    i = pl.program_id(0)
    hl = hl_ref[...]
    s = agg_ref[0] + agg_ref[1] + hl
    h = jnp.dot(s, w_ref[...], preferred_element_type=jnp.float32) + b_ref[...]
    h_ref[...] = _ln(h, g_ref[...], be_ref[...])
    onehot = (bt_ref[...] == lax.broadcasted_iota(jnp.int32, (1, G), 1)
              ).astype(jnp.float32)
    contrib = lax.dot_general(onehot, hl, (((0,), (0,)), ((), ())),
                              preferred_element_type=jnp.float32)

    @pl.when(i == 0)
    def _():
        pool_ref[...] = contrib

    @pl.when(i != 0)
    def _():
        pool_ref[...] += contrib


def _layer_update(agg, hl, batch2d, w, b, g, be):
    return pl.pallas_call(
        _layer_body,
        grid=(N // BN,),
        in_specs=[
            pl.BlockSpec((2, BN, H), lambda i: (0, i, 0)),
            pl.BlockSpec((BN, H), lambda i: (i, 0)),
            pl.BlockSpec((BN, 1), lambda i: (i, 0)),
            pl.BlockSpec((H, H), lambda i: (0, 0)),
            pl.BlockSpec((1, H), lambda i: (0, 0)),
            pl.BlockSpec((1, H), lambda i: (0, 0)),
            pl.BlockSpec((1, H), lambda i: (0, 0)),
        ],
        out_specs=[
            pl.BlockSpec((BN, H), lambda i: (i, 0)),
            pl.BlockSpec((G, H), lambda i: (0, 0)),
        ],
        out_shape=[
            jax.ShapeDtypeStruct((N, H), jnp.float32),
            jax.ShapeDtypeStruct((G, H), jnp.float32),
        ],
    )(agg, hl, batch2d, w, b.reshape(1, H), g.reshape(1, H),
      be.reshape(1, H))


def _vn_hl_body(pool_ref, vn_ref, w1_ref, b1_ref, w2_ref, b2_ref,
                h_ref, bt_ref, vno_ref, hlo_ref, vns_ref):
    i = pl.program_id(0)

    @pl.when(i == 0)
    def _():
        t = pool_ref[...] + vn_ref[...]
        t = jnp.maximum(jnp.dot(t, w1_ref[...],
                                preferred_element_type=jnp.float32)
                        + b1_ref[...], 0.0)
        t = jnp.maximum(jnp.dot(t, w2_ref[...],
                                preferred_element_type=jnp.float32)
                        + b2_ref[...], 0.0)
        vns_ref[...] = t
        vno_ref[...] = t

    onehot = (bt_ref[...] == lax.broadcasted_iota(jnp.int32, (1, G), 1)
              ).astype(jnp.float32)
    hlo_ref[...] = h_ref[...] + jnp.dot(onehot, vns_ref[...],
                                        preferred_element_type=jnp.float32)


def _vn_hl(pool, vn, w1, b1, w2, b2, h, batch2d):
    return pl.pallas_call(
        _vn_hl_body,
        grid=(N // BN,),
        in_specs=[
            pl.BlockSpec((G, H), lambda i: (0, 0)),
            pl.BlockSpec((G, H), lambda i: (0, 0)),
            pl.BlockSpec((H, H), lambda i: (0, 0)),
            pl.BlockSpec((1, H), lambda i: (0, 0)),
            pl.BlockSpec((H, H), lambda i: (0, 0)),
            pl.BlockSpec((1, H), lambda i: (0, 0)),
            pl.BlockSpec((BN, H), lambda i: (i, 0)),
            pl.BlockSpec((BN, 1), lambda i: (i, 0)),
        ],
        out_specs=[
            pl.BlockSpec((G, H), lambda i: (0, 0)),
            pl.BlockSpec((BN, H), lambda i: (i, 0)),
        ],
        out_shape=[
            jax.ShapeDtypeStruct((G, H), jnp.float32),
            jax.ShapeDtypeStruct((N, H), jnp.float32),
        ],
        scratch_shapes=[pltpu.VMEM((G, H), jnp.float32)],
    )(pool, vn, w1, b1.reshape(1, H), w2, b2.reshape(1, H), h, batch2d)


def _final_body(agg_ref, hl_ref, w_ref, b_ref, g_ref, be_ref,
                wo_ref, bo_ref, o_ref):
    s = agg_ref[0] + agg_ref[1] + hl_ref[...]
    h = jnp.dot(s, w_ref[...], preferred_element_type=jnp.float32) + b_ref[...]
    h = _ln(h, g_ref[...], be_ref[...])
    o_ref[...] = jnp.maximum(
        jnp.dot(h, wo_ref[...], preferred_element_type=jnp.float32)
        + bo_ref[...], 0.0)


def _final(agg, hl, w, b, g, be, wo, bo):
    return pl.pallas_call(
        _final_body,
        grid=(N // BN,),
        in_specs=[
            pl.BlockSpec((2, BN, H), lambda i: (0, i, 0)),
            pl.BlockSpec((BN, H), lambda i: (i, 0)),
            pl.BlockSpec((H, H), lambda i: (0, 0)),
            pl.BlockSpec((1, H), lambda i: (0, 0)),
            pl.BlockSpec((1, H), lambda i: (0, 0)),
            pl.BlockSpec((1, H), lambda i: (0, 0)),
            pl.BlockSpec((H, H), lambda i: (0, 0)),
            pl.BlockSpec((1, H), lambda i: (0, 0)),
        ],
        out_specs=pl.BlockSpec((BN, H), lambda i: (i, 0)),
        out_shape=jax.ShapeDtypeStruct((N, H), jnp.float32),
    )(agg, hl, w, b.reshape(1, H), g.reshape(1, H), be.reshape(1, H),
      wo, bo.reshape(1, H))


def _sc_edge_body(h_hbm, e_hbm, src_hbm, dst_hbm, out_hbm,
                  srcv, dstv, ev, rows, acc,
                  sg0, sg1, ssd0, ssd1, se0, se1):
    cid = lax.axis_index("c")
    sid = lax.axis_index("s")
    w = sid * 2 + cid
    sem_g = (sg0, sg1)
    sem_sd = (ssd0, ssd1)
    sem_e = (se0, se1)
    EH = EC // 2

    def issue_sd(j, b):
        g = j * NW + w
        pltpu.async_copy(src_hbm.at[g], srcv.at[b], sem_sd[b])
        pltpu.async_copy(dst_hbm.at[g], dstv.at[b], sem_sd[b])

    def wait_sd(j, b):
        g = j * NW + w
        pltpu.make_async_copy(src_hbm.at[g], srcv.at[b], sem_sd[b]).wait()
        pltpu.make_async_copy(dst_hbm.at[g], dstv.at[b], sem_sd[b]).wait()

    def issue_e(j, b):
        g = j * NW + w
        pltpu.async_copy(e_hbm.at[pl.ds(g * EH, EH)], ev.at[b], sem_e[b])

    def wait_e(j, b):
        g = j * NW + w
        pltpu.make_async_copy(e_hbm.at[pl.ds(g * EH, EH)], ev.at[b],
                              sem_e[b]).wait()

    def issue_gather(b):
        pltpu.async_copy(h_hbm.at[srcv.at[b, 0]], rows.at[b], sem_g[b])

    def wait_gather(b):
        pltpu.make_async_copy(h_hbm.at[srcv.at[b, 0]], rows.at[b],
                              sem_g[b]).wait()

    def compute(b):
        def _row(rr, cc):
            r0 = 2 * rr
            r1 = 2 * rr + 1
            for q in range(8):
                sq = pl.ds(16 * q, 16)
                wv = ev[b, rr, sq]
                lo = lax.bitcast_convert_type(wv << 16, jnp.float32)
                hi = lax.bitcast_convert_type(
                    wv & jnp.uint32(0xFFFF0000), jnp.float32)
                rows[b, r0, sq] = jnp.maximum(rows[b, r0, sq] + lo, 0.0)
                rows[b, r1, sq] = jnp.maximum(rows[b, r1, sq] + hi, 0.0)
            return cc

        lax.fori_loop(0, EH, _row, 0)

    def scatter(b):
        pltpu.sync_copy(rows.at[b], acc.at[dstv.at[b, 0]], add=True)

    def head(j, b):
        wait_e(j, b)
        wait_gather(b)

    def stage(j, b):
        wait_sd(j + 1, 1 - b)
        issue_gather(1 - b)

    def tail(j, b):
        compute(b)
        scatter(b)

    def _zrow(r, c):
        for jj in range(8):
            rows[0, r, pl.ds(jj * 16, 16)] = jnp.zeros((16,), jnp.float32)
        return c

    lax.fori_loop(0, EC, _zrow, 0)
    for t in range(RPS // RCP):
        pltpu.sync_copy(rows.at[0, pl.ds(0, RCP)],
                        acc.at[pl.ds(sid * RPS + t * RCP, RCP)])

    @pl.when(sid == 0)
    def _():
        pltpu.sync_copy(rows.at[0, pl.ds(0, RTAIL)],
                        acc.at[pl.ds(16 * RPS, RTAIL)])

    plsc.subcore_barrier()

    issue_sd(0, 0)
    issue_sd(1, 1)
    issue_e(0, 0)
    issue_e(1, 1)
    wait_sd(0, 0)
    issue_gather(0)

    def _pair(jo, c):
        for b in (0, 1):
            j = 2 * jo + b
            head(j, b)
            stage(j, b)
            tail(j, b)
            issue_sd(j + 2, b)
            issue_e(j + 2, b)
        return c

    lax.fori_loop(0, (CPW - 3) // 2, _pair, 0)
    for j in range(2 * ((CPW - 3) // 2), CPW):
        b = j & 1
        head(j, b)
        if j + 1 < CPW:
            stage(j, b)
        tail(j, b)
        if j + 2 < CPW:
            issue_sd(j + 2, b)
            issue_e(j + 2, b)
    plsc.subcore_barrier()

    for t in range(RPS // RCP):
        base = sid * RPS + t * RCP
        pltpu.sync_copy(acc.at[pl.ds(base, RCP)], rows.at[0, pl.ds(0, RCP)])
        pltpu.sync_copy(rows.at[0, pl.ds(0, RCP)],
                        out_hbm.at[cid, pl.ds(base, RCP)])

    @pl.when(sid == 0)
    def _():
        pltpu.sync_copy(acc.at[pl.ds(16 * RPS, RTAIL)],
                        rows.at[0, pl.ds(0, RTAIL)])
        pltpu.sync_copy(rows.at[0, pl.ds(0, RTAIL)],
                        out_hbm.at[cid, pl.ds(16 * RPS, RTAIL)])


@functools.cache
def _sc_edge_kernel():
    return pl.kernel(
        _sc_edge_body,
        out_type=jax.ShapeDtypeStruct((2, N, H), jnp.float32),
        mesh=plsc.VectorSubcoreMesh(core_axis_name="c",
                                    subcore_axis_name="s"),
        scratch_types=[
            pltpu.VMEM((2, 1, EC), jnp.int32),
            pltpu.VMEM((2, 1, EC), jnp.int32),
            pltpu.VMEM((2, EC // 2, H), jnp.uint32),
            pltpu.VMEM((2, EC, H), jnp.float32),
            pltpu.VMEM_SHARED((N + 128, H), jnp.float32),
            pltpu.SemaphoreType.DMA,
            pltpu.SemaphoreType.DMA,
            pltpu.SemaphoreType.DMA,
            pltpu.SemaphoreType.DMA,
            pltpu.SemaphoreType.DMA,
            pltpu.SemaphoreType.DMA,
        ],
    )


def _sc_edge(hl, e, src2d, dst2d):
    return _sc_edge_kernel()(hl, e, src2d, dst2d)


@jax.jit
def _run(x, edge_index, edge_attr, edge_attr_v2, batch, W_init, b_init,
         We1, be1, We2, be2, W_layers, b_layers, gamma, beta,
         Wvn1, bvn1, Wvn2, bvn2, W_out, b_out):
    npad = EPAD - E
    src2d = jnp.concatenate(
        [edge_index[0], jnp.zeros((npad,), jnp.int32)]).reshape(NCHUNK, 1, EC)
    dst2d = jnp.concatenate(
        [edge_index[1],
         N + (jnp.arange(npad, dtype=jnp.int32) % 128)]).reshape(NCHUNK, 1, EC)
    batch2d = batch.reshape(N, 1)

    e = _edge_embed(edge_attr, edge_attr_v2, We1, We2, be1 + be2)
    hl = _node_init(x, W_init, b_init)
    vn = jnp.zeros((G, H), jnp.float32)

    for l in range(L - 1):
        agg = _sc_edge(hl, e, src2d, dst2d)
        h, pool = _layer_update(agg, hl, batch2d, W_layers[l], b_layers[l],
                                gamma[l], beta[l])
        vn, hl = _vn_hl(pool, vn, Wvn1[l], bvn1[l], Wvn2[l], bvn2[l],
                        h, batch2d)

    agg = _sc_edge(hl, e, src2d, dst2d)
    return _final(agg, hl, W_layers[L - 1], b_layers[L - 1],
                  gamma[L - 1], beta[L - 1], W_out, b_out)


def kernel(x, edge_index, edge_attr, edge_attr_v2, batch, W_init, b_init,
           We1, be1, We2, be2, W_layers, b_layers, gamma, beta,
           Wvn1, bvn1, Wvn2, bvn2, W_out, b_out):
    return _run(x, edge_index, edge_attr, edge_attr_v2, batch, W_init,
                b_init, We1, be1, We2, be2, W_layers, b_layers, gamma,
                beta, Wvn1, bvn1, Wvn2, bvn2, W_out, b_out)


# speedup vs baseline: 1.3149x; 1.3143x over previous
import functools

import jax
import jax.numpy as jnp
from jax import lax
from jax.experimental import pallas as pl
from jax.experimental.pallas import tpu as pltpu, tpu_sc as plsc

N = 10000
E = 320000
D = 128
H = 128
DE = 16
L = 4
G = 64

BN = 2000
BE = 8000
EC = 128
NW = 32
CPW = (E // EC + NW - 1) // NW
NCHUNK = CPW * NW
EPAD = NCHUNK * EC
RPS = 624
RCP = 104
RTAIL = N - 16 * RPS


def _init_body(x_ref, w_ref, b_ref, o_ref):
    o_ref[...] = jnp.dot(x_ref[...], w_ref[...],
                         preferred_element_type=jnp.float32) + b_ref[...]


def _node_init(x, w, b):
    return pl.pallas_call(
        _init_body,
        grid=(N // BN,),
        in_specs=[
            pl.BlockSpec((BN, D), lambda i: (i, 0)),
            pl.BlockSpec((D, H), lambda i: (0, 0)),
            pl.BlockSpec((1, H), lambda i: (0, 0)),
        ],
        out_specs=pl.BlockSpec((BN, H), lambda i: (i, 0)),
        out_shape=jax.ShapeDtypeStruct((N, H), jnp.float32),
    )(x, w, b.reshape(1, H))


def _edge_body(a1_ref, a2_ref, w1_ref, w2_ref, b_ref, o_ref):
    o_ref[...] = (jnp.dot(a1_ref[...], w1_ref[...],
                          preferred_element_type=jnp.float32)
                  + jnp.dot(a2_ref[...], w2_ref[...],
                            preferred_element_type=jnp.float32)
                  + b_ref[...])


def _edge_embed(ea, ea2, w1, w2, b12):
    return pl.pallas_call(
        _edge_body,
        grid=(E // BE,),
        in_specs=[
            pl.BlockSpec((BE, DE), lambda i: (i, 0)),
            pl.BlockSpec((BE, DE), lambda i: (i, 0)),
            pl.BlockSpec((DE, H), lambda i: (0, 0)),
            pl.BlockSpec((DE, H), lambda i: (0, 0)),
            pl.BlockSpec((1, H), lambda i: (0, 0)),
        ],
        out_specs=pl.BlockSpec((BE, H), lambda i: (i, 0)),
        out_shape=jax.ShapeDtypeStruct((EPAD, H), jnp.float32),
    )(ea, ea2, w1, w2, b12.reshape(1, H))


def _ln(h, g, b):
    mu = jnp.mean(h, axis=-1, keepdims=True)
    d = h - mu
    var = jnp.mean(d * d, axis=-1, keepdims=True)
    return d * jax.lax.rsqrt(var + 1e-5) * g + b


def _layer_body(agg_ref, hl_ref, bt_ref, w_ref, b_ref, g_ref, be_ref,
                h_ref, pool_ref):
    i = pl.program_id(0)
    hl = hl_ref[...]
    s = agg_ref[0] + agg_ref[1] + hl
    h = jnp.dot(s, w_ref[...], preferred_element_type=jnp.float32) + b_ref[...]
    h_ref[...] = _ln(h, g_ref[...], be_ref[...])
    onehot = (bt_ref[...] == lax.broadcasted_iota(jnp.int32, (1, G), 1)
              ).astype(jnp.float32)
    contrib = lax.dot_general(onehot, hl, (((0,), (0,)), ((), ())),
                              preferred_element_type=jnp.float32)

    @pl.when(i == 0)
    def _():
        pool_ref[...] = contrib

    @pl.when(i != 0)
    def _():
        pool_ref[...] += contrib


def _layer_update(agg, hl, batch2d, w, b, g, be):
    return pl.pallas_call(
        _layer_body,
        grid=(N // BN,),
        in_specs=[
            pl.BlockSpec((2, BN, H), lambda i: (0, i, 0)),
            pl.BlockSpec((BN, H), lambda i: (i, 0)),
            pl.BlockSpec((BN, 1), lambda i: (i, 0)),
            pl.BlockSpec((H, H), lambda i: (0, 0)),
            pl.BlockSpec((1, H), lambda i: (0, 0)),
            pl.BlockSpec((1, H), lambda i: (0, 0)),
            pl.BlockSpec((1, H), lambda i: (0, 0)),
        ],
        out_specs=[
            pl.BlockSpec((BN, H), lambda i: (i, 0)),
            pl.BlockSpec((G, H), lambda i: (0, 0)),
        ],
        out_shape=[
            jax.ShapeDtypeStruct((N, H), jnp.float32),
            jax.ShapeDtypeStruct((G, H), jnp.float32),
        ],
    )(agg, hl, batch2d, w, b.reshape(1, H), g.reshape(1, H),
      be.reshape(1, H))


def _vn_hl_body(pool_ref, vn_ref, w1_ref, b1_ref, w2_ref, b2_ref,
                h_ref, bt_ref, vno_ref, hlo_ref, vns_ref):
    i = pl.program_id(0)

    @pl.when(i == 0)
    def _():
        t = pool_ref[...] + vn_ref[...]
        t = jnp.maximum(jnp.dot(t, w1_ref[...],
                                preferred_element_type=jnp.float32)
                        + b1_ref[...], 0.0)
        t = jnp.maximum(jnp.dot(t, w2_ref[...],
                                preferred_element_type=jnp.float32)
                        + b2_ref[...], 0.0)
        vns_ref[...] = t
        vno_ref[...] = t

    onehot = (bt_ref[...] == lax.broadcasted_iota(jnp.int32, (1, G), 1)
              ).astype(jnp.float32)
    hlo_ref[...] = h_ref[...] + jnp.dot(onehot, vns_ref[...],
                                        preferred_element_type=jnp.float32)


def _vn_hl(pool, vn, w1, b1, w2, b2, h, batch2d):
    return pl.pallas_call(
        _vn_hl_body,
        grid=(N // BN,),
        in_specs=[
            pl.BlockSpec((G, H), lambda i: (0, 0)),
            pl.BlockSpec((G, H), lambda i: (0, 0)),
            pl.BlockSpec((H, H), lambda i: (0, 0)),
            pl.BlockSpec((1, H), lambda i: (0, 0)),
            pl.BlockSpec((H, H), lambda i: (0, 0)),
            pl.BlockSpec((1, H), lambda i: (0, 0)),
            pl.BlockSpec((BN, H), lambda i: (i, 0)),
            pl.BlockSpec((BN, 1), lambda i: (i, 0)),
        ],
        out_specs=[
            pl.BlockSpec((G, H), lambda i: (0, 0)),
            pl.BlockSpec((BN, H), lambda i: (i, 0)),
        ],
        out_shape=[
            jax.ShapeDtypeStruct((G, H), jnp.float32),
            jax.ShapeDtypeStruct((N, H), jnp.float32),
        ],
        scratch_shapes=[pltpu.VMEM((G, H), jnp.float32)],
    )(pool, vn, w1, b1.reshape(1, H), w2, b2.reshape(1, H), h, batch2d)


def _final_body(agg_ref, hl_ref, w_ref, b_ref, g_ref, be_ref,
                wo_ref, bo_ref, o_ref):
    s = agg_ref[0] + agg_ref[1] + hl_ref[...]
    h = jnp.dot(s, w_ref[...], preferred_element_type=jnp.float32) + b_ref[...]
    h = _ln(h, g_ref[...], be_ref[...])
    o_ref[...] = jnp.maximum(
        jnp.dot(h, wo_ref[...], preferred_element_type=jnp.float32)
        + bo_ref[...], 0.0)


def _final(agg, hl, w, b, g, be, wo, bo):
    return pl.pallas_call(
        _final_body,
        grid=(N // BN,),
        in_specs=[
            pl.BlockSpec((2, BN, H), lambda i: (0, i, 0)),
            pl.BlockSpec((BN, H), lambda i: (i, 0)),
            pl.BlockSpec((H, H), lambda i: (0, 0)),
            pl.BlockSpec((1, H), lambda i: (0, 0)),
            pl.BlockSpec((1, H), lambda i: (0, 0)),
            pl.BlockSpec((1, H), lambda i: (0, 0)),
            pl.BlockSpec((H, H), lambda i: (0, 0)),
            pl.BlockSpec((1, H), lambda i: (0, 0)),
        ],
        out_specs=pl.BlockSpec((BN, H), lambda i: (i, 0)),
        out_shape=jax.ShapeDtypeStruct((N, H), jnp.float32),
    )(agg, hl, w, b.reshape(1, H), g.reshape(1, H), be.reshape(1, H),
      wo, bo.reshape(1, H))


def _sc_edge_body(h_hbm, e_hbm, src_hbm, dst_hbm, out_hbm,
                  srcv, dstv, ev, rows, acc,
                  sg0, sg1, ssd0, ssd1, se):
    cid = lax.axis_index("c")
    sid = lax.axis_index("s")
    w = sid * 2 + cid
    sem_g = (sg0, sg1)
    sem_sd = (ssd0, ssd1)

    def issue_sd(j):
        b = j & 1
        g = j * NW + w
        pltpu.async_copy(src_hbm.at[g], srcv.at[b], sem_sd[b])
        pltpu.async_copy(dst_hbm.at[g], dstv.at[b], sem_sd[b])

    def wait_sd(j):
        b = j & 1
        g = j * NW + w
        pltpu.make_async_copy(src_hbm.at[g], srcv.at[b], sem_sd[b]).wait()
        pltpu.make_async_copy(dst_hbm.at[g], dstv.at[b], sem_sd[b]).wait()

    def issue_e(j):
        g = j * NW + w
        pltpu.async_copy(e_hbm.at[pl.ds(g * EC, EC)], ev, se)

    def wait_e(j):
        g = j * NW + w
        pltpu.make_async_copy(e_hbm.at[pl.ds(g * EC, EC)], ev, se).wait()

    def issue_gather(j):
        b = j & 1
        pltpu.async_copy(h_hbm.at[srcv.at[b, 0]], rows.at[b], sem_g[b])

    def wait_gather(j):
        b = j & 1
        pltpu.make_async_copy(h_hbm.at[srcv.at[b, 0]], rows.at[b],
                              sem_g[b]).wait()

    def compute(j):
        b = j & 1

        def _row(r, cc):
            for jj in range(8):
                sl = pl.ds(jj * 16, 16)
                v = rows[b, r, sl] + ev[r, sl]
                rows[b, r, sl] = jnp.maximum(v, 0.0)
            return cc

        lax.fori_loop(0, EC, _row, 0)

    def scatter(j):
        b = j & 1
        pltpu.sync_copy(rows.at[b], acc.at[dstv.at[b, 0]], add=True)

    def _zrow(r, c):
        for jj in range(8):
            rows[0, r, pl.ds(jj * 16, 16)] = jnp.zeros((16,), jnp.float32)
        return c

    lax.fori_loop(0, EC, _zrow, 0)
    for t in range(RPS // RCP):
        pltpu.sync_copy(rows.at[0, pl.ds(0, RCP)],
                        acc.at[pl.ds(sid * RPS + t * RCP, RCP)])

    @pl.when(sid == 0)
    def _():
        pltpu.sync_copy(rows.at[0, pl.ds(0, RTAIL)],
                        acc.at[pl.ds(16 * RPS, RTAIL)])

    plsc.subcore_barrier()

    issue_sd(0)
    issue_sd(1)
    issue_e(0)
    wait_sd(0)
    issue_gather(0)
    for j in range(CPW):
        wait_e(j)
        wait_gather(j)
        if j + 1 < CPW:
            wait_sd(j + 1)
            issue_gather(j + 1)
        compute(j)
        if j + 1 < CPW:
            issue_e(j + 1)
        scatter(j)
        if j + 2 < CPW:
            issue_sd(j + 2)
    plsc.subcore_barrier()

    for t in range(RPS // RCP):
        base = sid * RPS + t * RCP
        pltpu.sync_copy(acc.at[pl.ds(base, RCP)], rows.at[0, pl.ds(0, RCP)])
        pltpu.sync_copy(rows.at[0, pl.ds(0, RCP)],
                        out_hbm.at[cid, pl.ds(base, RCP)])

    @pl.when(sid == 0)
    def _():
        pltpu.sync_copy(acc.at[pl.ds(16 * RPS, RTAIL)],
                        rows.at[0, pl.ds(0, RTAIL)])
        pltpu.sync_copy(rows.at[0, pl.ds(0, RTAIL)],
                        out_hbm.at[cid, pl.ds(16 * RPS, RTAIL)])


@functools.cache
def _sc_edge_kernel():
    return pl.kernel(
        _sc_edge_body,
        out_type=jax.ShapeDtypeStruct((2, N, H), jnp.float32),
        mesh=plsc.VectorSubcoreMesh(core_axis_name="c",
                                    subcore_axis_name="s"),
        scratch_types=[
            pltpu.VMEM((2, 1, EC), jnp.int32),
            pltpu.VMEM((2, 1, EC), jnp.int32),
            pltpu.VMEM((EC, H), jnp.float32),
            pltpu.VMEM((2, EC, H), jnp.float32),
            pltpu.VMEM_SHARED((N + 128, H), jnp.float32),
            pltpu.SemaphoreType.DMA,
            pltpu.SemaphoreType.DMA,
            pltpu.SemaphoreType.DMA,
            pltpu.SemaphoreType.DMA,
            pltpu.SemaphoreType.DMA,
        ],
    )


def _sc_edge(hl, e, src2d, dst2d):
    return _sc_edge_kernel()(hl, e, src2d, dst2d)


@jax.jit
def _run(x, edge_index, edge_attr, edge_attr_v2, batch, W_init, b_init,
         We1, be1, We2, be2, W_layers, b_layers, gamma, beta,
         Wvn1, bvn1, Wvn2, bvn2, W_out, b_out):
    npad = EPAD - E
    src2d = jnp.concatenate(
        [edge_index[0], jnp.zeros((npad,), jnp.int32)]).reshape(NCHUNK, 1, EC)
    dst2d = jnp.concatenate(
        [edge_index[1],
         N + (jnp.arange(npad, dtype=jnp.int32) % 128)]).reshape(NCHUNK, 1, EC)
    batch2d = batch.reshape(N, 1)

    e = _edge_embed(edge_attr, edge_attr_v2, We1, We2, be1 + be2)
    hl = _node_init(x, W_init, b_init)
    vn = jnp.zeros((G, H), jnp.float32)

    for l in range(L - 1):
        agg = _sc_edge(hl, e, src2d, dst2d)
        h, pool = _layer_update(agg, hl, batch2d, W_layers[l], b_layers[l],
                                gamma[l], beta[l])
        vn, hl = _vn_hl(pool, vn, Wvn1[l], bvn1[l], Wvn2[l], bvn2[l],
                        h, batch2d)

    agg = _sc_edge(hl, e, src2d, dst2d)
    return _final(agg, hl, W_layers[L - 1], b_layers[L - 1],
                  gamma[L - 1], beta[L - 1], W_out, b_out)


def kernel(x, edge_index, edge_attr, edge_attr_v2, batch, W_init, b_init,
           We1, be1, We2, be2, W_layers, b_layers, gamma, beta,
           Wvn1, bvn1, Wvn2, bvn2, W_out, b_out):
    return _run(x, edge_index, edge_attr, edge_attr_v2, batch, W_init,
                b_init, We1, be1, We2, be2, W_layers, b_layers, gamma,
                beta, Wvn1, bvn1, Wvn2, bvn2, W_out, b_out)


# speedup vs baseline: 1.8661x; 1.4192x over previous
import functools

import jax
import jax.numpy as jnp
from jax import lax
from jax.experimental import pallas as pl
from jax.experimental.pallas import tpu as pltpu, tpu_sc as plsc

N = 10000
E = 320000
D = 128
H = 128
DE = 16
L = 4
G = 64

BN = 2000
BE = 8000
EC = 128
NCHUNK = E // EC
NW = 32
CPW = (NCHUNK + NW - 1) // NW
RPS = 624
RCP = 104
RTAIL = N - 16 * RPS


def _init_body(x_ref, w_ref, b_ref, o_ref):
    o_ref[...] = jnp.dot(x_ref[...], w_ref[...],
                         preferred_element_type=jnp.float32) + b_ref[...]


def _node_init(x, w, b):
    return pl.pallas_call(
        _init_body,
        grid=(N // BN,),
        in_specs=[
            pl.BlockSpec((BN, D), lambda i: (i, 0)),
            pl.BlockSpec((D, H), lambda i: (0, 0)),
            pl.BlockSpec((1, H), lambda i: (0, 0)),
        ],
        out_specs=pl.BlockSpec((BN, H), lambda i: (i, 0)),
        out_shape=jax.ShapeDtypeStruct((N, H), jnp.float32),
    )(x, w, b.reshape(1, H))


def _edge_body(a1_ref, a2_ref, w1_ref, w2_ref, b_ref, o_ref):
    o_ref[...] = (jnp.dot(a1_ref[...], w1_ref[...],
                          preferred_element_type=jnp.float32)
                  + jnp.dot(a2_ref[...], w2_ref[...],
                            preferred_element_type=jnp.float32)
                  + b_ref[...])


def _edge_embed(ea, ea2, w1, w2, b12):
    return pl.pallas_call(
        _edge_body,
        grid=(E // BE,),
        in_specs=[
            pl.BlockSpec((BE, DE), lambda i: (i, 0)),
            pl.BlockSpec((BE, DE), lambda i: (i, 0)),
            pl.BlockSpec((DE, H), lambda i: (0, 0)),
            pl.BlockSpec((DE, H), lambda i: (0, 0)),
            pl.BlockSpec((1, H), lambda i: (0, 0)),
        ],
        out_specs=pl.BlockSpec((BE, H), lambda i: (i, 0)),
        out_shape=jax.ShapeDtypeStruct((E, H), jnp.float32),
    )(ea, ea2, w1, w2, b12.reshape(1, H))


def _ln(h, g, b):
    mu = jnp.mean(h, axis=-1, keepdims=True)
    d = h - mu
    var = jnp.mean(d * d, axis=-1, keepdims=True)
    return d * jax.lax.rsqrt(var + 1e-5) * g + b


def _layer_body(agg_ref, hl_ref, bt_ref, w_ref, b_ref, g_ref, be_ref,
                h_ref, pool_ref):
    i = pl.program_id(0)
    hl = hl_ref[...]
    s = agg_ref[0] + agg_ref[1] + hl
    h = jnp.dot(s, w_ref[...], preferred_element_type=jnp.float32) + b_ref[...]
    h_ref[...] = _ln(h, g_ref[...], be_ref[...])
    onehot = (bt_ref[...] == lax.broadcasted_iota(jnp.int32, (1, G), 1)
              ).astype(jnp.float32)
    contrib = lax.dot_general(onehot, hl, (((0,), (0,)), ((), ())),
                              preferred_element_type=jnp.float32)

    @pl.when(i == 0)
    def _():
        pool_ref[...] = contrib

    @pl.when(i != 0)
    def _():
        pool_ref[...] += contrib


def _layer_update(agg, hl, batch2d, w, b, g, be):
    return pl.pallas_call(
        _layer_body,
        grid=(N // BN,),
        in_specs=[
            pl.BlockSpec((2, BN, H), lambda i: (0, i, 0)),
            pl.BlockSpec((BN, H), lambda i: (i, 0)),
            pl.BlockSpec((BN, 1), lambda i: (i, 0)),
            pl.BlockSpec((H, H), lambda i: (0, 0)),
            pl.BlockSpec((1, H), lambda i: (0, 0)),
            pl.BlockSpec((1, H), lambda i: (0, 0)),
            pl.BlockSpec((1, H), lambda i: (0, 0)),
        ],
        out_specs=[
            pl.BlockSpec((BN, H), lambda i: (i, 0)),
            pl.BlockSpec((G, H), lambda i: (0, 0)),
        ],
        out_shape=[
            jax.ShapeDtypeStruct((N, H), jnp.float32),
            jax.ShapeDtypeStruct((G, H), jnp.float32),
        ],
    )(agg, hl, batch2d, w, b.reshape(1, H), g.reshape(1, H),
      be.reshape(1, H))


def _vn_hl_body(pool_ref, vn_ref, w1_ref, b1_ref, w2_ref, b2_ref,
                h_ref, bt_ref, vno_ref, hlo_ref, vns_ref):
    i = pl.program_id(0)

    @pl.when(i == 0)
    def _():
        t = pool_ref[...] + vn_ref[...]
        t = jnp.maximum(jnp.dot(t, w1_ref[...],
                                preferred_element_type=jnp.float32)
                        + b1_ref[...], 0.0)
        t = jnp.maximum(jnp.dot(t, w2_ref[...],
                                preferred_element_type=jnp.float32)
                        + b2_ref[...], 0.0)
        vns_ref[...] = t
        vno_ref[...] = t

    onehot = (bt_ref[...] == lax.broadcasted_iota(jnp.int32, (1, G), 1)
              ).astype(jnp.float32)
    hlo_ref[...] = h_ref[...] + jnp.dot(onehot, vns_ref[...],
                                        preferred_element_type=jnp.float32)


def _vn_hl(pool, vn, w1, b1, w2, b2, h, batch2d):
    return pl.pallas_call(
        _vn_hl_body,
        grid=(N // BN,),
        in_specs=[
            pl.BlockSpec((G, H), lambda i: (0, 0)),
            pl.BlockSpec((G, H), lambda i: (0, 0)),
            pl.BlockSpec((H, H), lambda i: (0, 0)),
            pl.BlockSpec((1, H), lambda i: (0, 0)),
            pl.BlockSpec((H, H), lambda i: (0, 0)),
            pl.BlockSpec((1, H), lambda i: (0, 0)),
            pl.BlockSpec((BN, H), lambda i: (i, 0)),
            pl.BlockSpec((BN, 1), lambda i: (i, 0)),
        ],
        out_specs=[
            pl.BlockSpec((G, H), lambda i: (0, 0)),
            pl.BlockSpec((BN, H), lambda i: (i, 0)),
        ],
        out_shape=[
            jax.ShapeDtypeStruct((G, H), jnp.float32),
            jax.ShapeDtypeStruct((N, H), jnp.float32),
        ],
        scratch_shapes=[pltpu.VMEM((G, H), jnp.float32)],
    )(pool, vn, w1, b1.reshape(1, H), w2, b2.reshape(1, H), h, batch2d)


def _final_body(agg_ref, hl_ref, w_ref, b_ref, g_ref, be_ref,
                wo_ref, bo_ref, o_ref):
    s = agg_ref[0] + agg_ref[1] + hl_ref[...]
    h = jnp.dot(s, w_ref[...], preferred_element_type=jnp.float32) + b_ref[...]
    h = _ln(h, g_ref[...], be_ref[...])
    o_ref[...] = jnp.maximum(
        jnp.dot(h, wo_ref[...], preferred_element_type=jnp.float32)
        + bo_ref[...], 0.0)


def _final(agg, hl, w, b, g, be, wo, bo):
    return pl.pallas_call(
        _final_body,
        grid=(N // BN,),
        in_specs=[
            pl.BlockSpec((2, BN, H), lambda i: (0, i, 0)),
            pl.BlockSpec((BN, H), lambda i: (i, 0)),
            pl.BlockSpec((H, H), lambda i: (0, 0)),
            pl.BlockSpec((1, H), lambda i: (0, 0)),
            pl.BlockSpec((1, H), lambda i: (0, 0)),
            pl.BlockSpec((1, H), lambda i: (0, 0)),
            pl.BlockSpec((H, H), lambda i: (0, 0)),
            pl.BlockSpec((1, H), lambda i: (0, 0)),
        ],
        out_specs=pl.BlockSpec((BN, H), lambda i: (i, 0)),
        out_shape=jax.ShapeDtypeStruct((N, H), jnp.float32),
    )(agg, hl, w, b.reshape(1, H), g.reshape(1, H), be.reshape(1, H),
      wo, bo.reshape(1, H))


def _sc_edge_body(h_hbm, e_hbm, src_hbm, dst_hbm, out_hbm,
                  srcv, dstv, ev, rows, acc,
                  sg0, sg1, ssd0, ssd1, se):
    cid = lax.axis_index("c")
    sid = lax.axis_index("s")
    w = sid * 2 + cid
    sem_g = (sg0, sg1)
    sem_sd = (ssd0, ssd1)
    LAST = CPW - 1
    NTAILW = NCHUNK - LAST * NW
    w_valid = w < NTAILW

    def guarded(j, fn):
        if j < LAST:
            fn()
        else:
            pl.when(w_valid)(lambda: (fn(), None)[1])

    def issue_sd(j):
        b = j & 1
        g = j * NW + w
        pltpu.async_copy(src_hbm.at[g], srcv.at[b], sem_sd[b])
        pltpu.async_copy(dst_hbm.at[g], dstv.at[b], sem_sd[b])

    def wait_sd(j):
        b = j & 1
        g = j * NW + w
        pltpu.make_async_copy(src_hbm.at[g], srcv.at[b], sem_sd[b]).wait()
        pltpu.make_async_copy(dst_hbm.at[g], dstv.at[b], sem_sd[b]).wait()

    def issue_e(j):
        g = j * NW + w
        pltpu.async_copy(e_hbm.at[pl.ds(g * EC, EC)], ev, se)

    def wait_e(j):
        g = j * NW + w
        pltpu.make_async_copy(e_hbm.at[pl.ds(g * EC, EC)], ev, se).wait()

    def issue_gather(j):
        b = j & 1
        pltpu.async_copy(h_hbm.at[srcv.at[b, 0]], rows.at[b], sem_g[b])

    def wait_gather(j):
        b = j & 1
        pltpu.make_async_copy(h_hbm.at[srcv.at[b, 0]], rows.at[b],
                              sem_g[b]).wait()

    def compute(j):
        b = j & 1

        def _row(r, cc):
            for jj in range(8):
                sl = pl.ds(jj * 16, 16)
                v = rows[b, r, sl] + ev[r, sl]
                rows[b, r, sl] = jnp.maximum(v, 0.0)
            return cc

        lax.fori_loop(0, EC, _row, 0)

    def scatter(j):
        b = j & 1
        pltpu.sync_copy(rows.at[b], acc.at[dstv.at[b, 0]], add=True)

    def _zrow(r, c):
        for jj in range(8):
            rows[0, r, pl.ds(jj * 16, 16)] = jnp.zeros((16,), jnp.float32)
        return c

    lax.fori_loop(0, EC, _zrow, 0)
    for t in range(RPS // RCP):
        pltpu.sync_copy(rows.at[0, pl.ds(0, RCP)],
                        acc.at[pl.ds(sid * RPS + t * RCP, RCP)])

    @pl.when(sid == 0)
    def _():
        pltpu.sync_copy(rows.at[0, pl.ds(0, RTAIL)],
                        acc.at[pl.ds(16 * RPS, RTAIL)])

    plsc.subcore_barrier()

    issue_sd(0)
    issue_sd(1)
    issue_e(0)
    wait_sd(0)
    issue_gather(0)
    for j in range(CPW):
        guarded(j, lambda j=j: (wait_e(j), wait_gather(j)))
        if j + 1 < CPW:
            guarded(j + 1, lambda j=j: (wait_sd(j + 1), issue_gather(j + 1)))
        guarded(j, lambda j=j: compute(j))
        if j + 1 < CPW:
            guarded(j + 1, lambda j=j: issue_e(j + 1))
        guarded(j, lambda j=j: scatter(j))
        if j + 2 < CPW:
            guarded(j + 2, lambda j=j: issue_sd(j + 2))
    plsc.subcore_barrier()

    for t in range(RPS // RCP):
        base = sid * RPS + t * RCP
        pltpu.sync_copy(acc.at[pl.ds(base, RCP)], rows.at[0, pl.ds(0, RCP)])
        pltpu.sync_copy(rows.at[0, pl.ds(0, RCP)],
                        out_hbm.at[cid, pl.ds(base, RCP)])

    @pl.when(sid == 0)
    def _():
        pltpu.sync_copy(acc.at[pl.ds(16 * RPS, RTAIL)],
                        rows.at[0, pl.ds(0, RTAIL)])
        pltpu.sync_copy(rows.at[0, pl.ds(0, RTAIL)],
                        out_hbm.at[cid, pl.ds(16 * RPS, RTAIL)])


@functools.cache
def _sc_edge_kernel():
    return pl.kernel(
        _sc_edge_body,
        out_type=jax.ShapeDtypeStruct((2, N, H), jnp.float32),
        mesh=plsc.VectorSubcoreMesh(core_axis_name="c",
                                    subcore_axis_name="s"),
        scratch_types=[
            pltpu.VMEM((2, 1, EC), jnp.int32),
            pltpu.VMEM((2, 1, EC), jnp.int32),
            pltpu.VMEM((EC, H), jnp.float32),
            pltpu.VMEM((2, EC, H), jnp.float32),
            pltpu.VMEM_SHARED((N, H), jnp.float32),
            pltpu.SemaphoreType.DMA,
            pltpu.SemaphoreType.DMA,
            pltpu.SemaphoreType.DMA,
            pltpu.SemaphoreType.DMA,
            pltpu.SemaphoreType.DMA,
        ],
    )


def _sc_edge(hl, e, src2d, dst2d):
    return _sc_edge_kernel()(hl, e, src2d, dst2d)


@jax.jit
def _run(x, edge_index, edge_attr, edge_attr_v2, batch, W_init, b_init,
         We1, be1, We2, be2, W_layers, b_layers, gamma, beta,
         Wvn1, bvn1, Wvn2, bvn2, W_out, b_out):
    src2d = edge_index[0].reshape(NCHUNK, 1, EC)
    dst2d = edge_index[1].reshape(NCHUNK, 1, EC)
    batch2d = batch.reshape(N, 1)

    e = _edge_embed(edge_attr, edge_attr_v2, We1, We2, be1 + be2)
    hl = _node_init(x, W_init, b_init)
    vn = jnp.zeros((G, H), jnp.float32)

    for l in range(L - 1):
        agg = _sc_edge(hl, e, src2d, dst2d)
        h, pool = _layer_update(agg, hl, batch2d, W_layers[l], b_layers[l],
                                gamma[l], beta[l])
        vn, hl = _vn_hl(pool, vn, Wvn1[l], bvn1[l], Wvn2[l], bvn2[l],
                        h, batch2d)

    agg = _sc_edge(hl, e, src2d, dst2d)
    return _final(agg, hl, W_layers[L - 1], b_layers[L - 1],
                  gamma[L - 1], beta[L - 1], W_out, b_out)


def kernel(x, edge_index, edge_attr, edge_attr_v2, batch, W_init, b_init,
           We1, be1, We2, be2, W_layers, b_layers, gamma, beta,
           Wvn1, bvn1, Wvn2, bvn2, W_out, b_out):
    return _run(x, edge_index, edge_attr, edge_attr_v2, batch, W_init,
                b_init, We1, be1, We2, be2, W_layers, b_layers, gamma,
                beta, Wvn1, bvn1, Wvn2, bvn2, W_out, b_out)


# speedup vs baseline: 1.8838x; 1.0095x over previous
import functools

import jax
import jax.numpy as jnp
from jax import lax
from jax.experimental import pallas as pl
from jax.experimental.pallas import tpu as pltpu, tpu_sc as plsc

N = 10000
E = 320000
D = 128
H = 128
DE = 16
L = 4
G = 64

BN = 2000
BE = 8000
EC = 128
NCHUNK = E // EC
NW = 32
CPW = (NCHUNK + NW - 1) // NW
RPS = 624
RCP = 104
RTAIL = N - 16 * RPS


def _init_body(x_ref, w_ref, b_ref, o_ref):
    o_ref[...] = jnp.dot(x_ref[...], w_ref[...],
                         preferred_element_type=jnp.float32) + b_ref[...]


def _node_init(x, w, b):
    return pl.pallas_call(
        _init_body,
        grid=(N // BN,),
        in_specs=[
            pl.BlockSpec((BN, D), lambda i: (i, 0)),
            pl.BlockSpec((D, H), lambda i: (0, 0)),
            pl.BlockSpec((1, H), lambda i: (0, 0)),
        ],
        out_specs=pl.BlockSpec((BN, H), lambda i: (i, 0)),
        out_shape=jax.ShapeDtypeStruct((N, H), jnp.float32),
    )(x, w, b.reshape(1, H))


def _edge_body(a1_ref, a2_ref, w1_ref, w2_ref, b_ref, o_ref):
    o_ref[...] = (jnp.dot(a1_ref[...], w1_ref[...],
                          preferred_element_type=jnp.float32)
                  + jnp.dot(a2_ref[...], w2_ref[...],
                            preferred_element_type=jnp.float32)
                  + b_ref[...])


def _edge_embed(ea, ea2, w1, w2, b12):
    return pl.pallas_call(
        _edge_body,
        grid=(E // BE,),
        in_specs=[
            pl.BlockSpec((BE, DE), lambda i: (i, 0)),
            pl.BlockSpec((BE, DE), lambda i: (i, 0)),
            pl.BlockSpec((DE, H), lambda i: (0, 0)),
            pl.BlockSpec((DE, H), lambda i: (0, 0)),
            pl.BlockSpec((1, H), lambda i: (0, 0)),
        ],
        out_specs=pl.BlockSpec((BE, H), lambda i: (i, 0)),
        out_shape=jax.ShapeDtypeStruct((E, H), jnp.float32),
    )(ea, ea2, w1, w2, b12.reshape(1, H))


def _ln(h, g, b):
    mu = jnp.mean(h, axis=-1, keepdims=True)
    d = h - mu
    var = jnp.mean(d * d, axis=-1, keepdims=True)
    return d * jax.lax.rsqrt(var + 1e-5) * g + b


def _fused_layer_body(agg_ref, hl_ref, bt_ref, vn_ref,
                      w_ref, b_ref, g_ref, be_ref,
                      w1_ref, b1_ref, w2_ref, b2_ref,
                      hlo_ref, vno_ref, hbuf, pool, vns):
    p = pl.program_id(0)
    i = pl.program_id(1)

    @pl.when(p == 0)
    def _():
        hl = hl_ref[...]
        sm = agg_ref[0] + agg_ref[1] + hl
        h = (jnp.dot(sm, w_ref[...], preferred_element_type=jnp.float32)
             + b_ref[...])
        hbuf[pl.ds(i * BN, BN), :] = _ln(h, g_ref[...], be_ref[...])
        onehot = (bt_ref[...] == lax.broadcasted_iota(jnp.int32, (1, G), 1)
                  ).astype(jnp.float32)
        contrib = lax.dot_general(onehot, hl, (((0,), (0,)), ((), ())),
                                  preferred_element_type=jnp.float32)

        @pl.when(i == 0)
        def _():
            pool[...] = contrib

        @pl.when(i != 0)
        def _():
            pool[...] += contrib

    @pl.when(p == 1)
    def _():
        @pl.when(i == 0)
        def _():
            t = pool[...] + vn_ref[...]
            t = jnp.maximum(jnp.dot(t, w1_ref[...],
                                    preferred_element_type=jnp.float32)
                            + b1_ref[...], 0.0)
            t = jnp.maximum(jnp.dot(t, w2_ref[...],
                                    preferred_element_type=jnp.float32)
                            + b2_ref[...], 0.0)
            vns[...] = t
            vno_ref[...] = t

        onehot = (bt_ref[...] == lax.broadcasted_iota(jnp.int32, (1, G), 1)
                  ).astype(jnp.float32)
        hlo_ref[...] = (hbuf[pl.ds(i * BN, BN), :]
                        + jnp.dot(onehot, vns[...],
                                  preferred_element_type=jnp.float32))


def _fused_layer(agg, hl, batch2d, vn, w, b, g, be, w1, b1, w2, b2):
    zero = lambda p, i: (0, 0)
    return pl.pallas_call(
        _fused_layer_body,
        grid=(2, N // BN),
        in_specs=[
            pl.BlockSpec((2, BN, H), lambda p, i: (0, (1 - p) * i, 0)),
            pl.BlockSpec((BN, H), lambda p, i: ((1 - p) * i, 0)),
            pl.BlockSpec((BN, 1), lambda p, i: (i, 0)),
            pl.BlockSpec((G, H), zero),
            pl.BlockSpec((H, H), zero),
            pl.BlockSpec((1, H), zero),
            pl.BlockSpec((1, H), zero),
            pl.BlockSpec((1, H), zero),
            pl.BlockSpec((H, H), zero),
            pl.BlockSpec((1, H), zero),
            pl.BlockSpec((H, H), zero),
            pl.BlockSpec((1, H), zero),
        ],
        out_specs=[
            pl.BlockSpec((BN, H), lambda p, i: (p * i, 0)),
            pl.BlockSpec((G, H), zero),
        ],
        out_shape=[
            jax.ShapeDtypeStruct((N, H), jnp.float32),
            jax.ShapeDtypeStruct((G, H), jnp.float32),
        ],
        scratch_shapes=[
            pltpu.VMEM((N, H), jnp.float32),
            pltpu.VMEM((G, H), jnp.float32),
            pltpu.VMEM((G, H), jnp.float32),
        ],
    )(agg, hl, batch2d, vn, w, b.reshape(1, H), g.reshape(1, H),
      be.reshape(1, H), w1, b1.reshape(1, H), w2, b2.reshape(1, H))


def _final_body(agg_ref, hl_ref, w_ref, b_ref, g_ref, be_ref,
                wo_ref, bo_ref, o_ref):
    s = agg_ref[0] + agg_ref[1] + hl_ref[...]
    h = jnp.dot(s, w_ref[...], preferred_element_type=jnp.float32) + b_ref[...]
    h = _ln(h, g_ref[...], be_ref[...])
    o_ref[...] = jnp.maximum(
        jnp.dot(h, wo_ref[...], preferred_element_type=jnp.float32)
        + bo_ref[...], 0.0)


def _final(agg, hl, w, b, g, be, wo, bo):
    return pl.pallas_call(
        _final_body,
        grid=(N // BN,),
        in_specs=[
            pl.BlockSpec((2, BN, H), lambda i: (0, i, 0)),
            pl.BlockSpec((BN, H), lambda i: (i, 0)),
            pl.BlockSpec((H, H), lambda i: (0, 0)),
            pl.BlockSpec((1, H), lambda i: (0, 0)),
            pl.BlockSpec((1, H), lambda i: (0, 0)),
            pl.BlockSpec((1, H), lambda i: (0, 0)),
            pl.BlockSpec((H, H), lambda i: (0, 0)),
            pl.BlockSpec((1, H), lambda i: (0, 0)),
        ],
        out_specs=pl.BlockSpec((BN, H), lambda i: (i, 0)),
        out_shape=jax.ShapeDtypeStruct((N, H), jnp.float32),
    )(agg, hl, w, b.reshape(1, H), g.reshape(1, H), be.reshape(1, H),
      wo, bo.reshape(1, H))


def _sc_edge_body(h_hbm, e_hbm, src_hbm, dst_hbm, out_hbm,
                  srcv, dstv, ev, rows, acc,
                  sg0, sg1, ssd0, ssd1, se):
    cid = lax.axis_index("c")
    sid = lax.axis_index("s")
    w = sid * 2 + cid
    sem_g = (sg0, sg1)
    sem_sd = (ssd0, ssd1)
    LAST = CPW - 1
    NTAILW = NCHUNK - LAST * NW
    w_valid = w < NTAILW

    def guarded(j, fn):
        if j < LAST:
            fn()
        else:
            pl.when(w_valid)(lambda: (fn(), None)[1])

    def issue_sd(j):
        b = j & 1
        g = j * NW + w
        pltpu.async_copy(src_hbm.at[g], srcv.at[b], sem_sd[b])
        pltpu.async_copy(dst_hbm.at[g], dstv.at[b], sem_sd[b])

    def wait_sd(j):
        b = j & 1
        g = j * NW + w
        pltpu.make_async_copy(src_hbm.at[g], srcv.at[b], sem_sd[b]).wait()
        pltpu.make_async_copy(dst_hbm.at[g], dstv.at[b], sem_sd[b]).wait()

    def issue_e(j):
        g = j * NW + w
        pltpu.async_copy(e_hbm.at[pl.ds(g * EC, EC)], ev, se)

    def wait_e(j):
        g = j * NW + w
        pltpu.make_async_copy(e_hbm.at[pl.ds(g * EC, EC)], ev, se).wait()

    def issue_gather(j):
        b = j & 1
        pltpu.async_copy(h_hbm.at[srcv.at[b, 0]], rows.at[b], sem_g[b])

    def wait_gather(j):
        b = j & 1
        pltpu.make_async_copy(h_hbm.at[srcv.at[b, 0]], rows.at[b],
                              sem_g[b]).wait()

    def compute(j):
        b = j & 1

        def _row(r, cc):
            for jj in range(8):
                sl = pl.ds(jj * 16, 16)
                v = rows[b, r, sl] + ev[r, sl]
                rows[b, r, sl] = jnp.maximum(v, 0.0)
            return cc

        lax.fori_loop(0, EC, _row, 0)

    def scatter(j):
        b = j & 1
        pltpu.sync_copy(rows.at[b], acc.at[dstv.at[b, 0]], add=True)

    def _zrow(r, c):
        for jj in range(8):
            rows[0, r, pl.ds(jj * 16, 16)] = jnp.zeros((16,), jnp.float32)
        return c

    lax.fori_loop(0, EC, _zrow, 0)
    for t in range(RPS // RCP):
        pltpu.sync_copy(rows.at[0, pl.ds(0, RCP)],
                        acc.at[pl.ds(sid * RPS + t * RCP, RCP)])

    @pl.when(sid == 0)
    def _():
        pltpu.sync_copy(rows.at[0, pl.ds(0, RTAIL)],
                        acc.at[pl.ds(16 * RPS, RTAIL)])

    plsc.subcore_barrier()

    issue_sd(0)
    issue_sd(1)
    issue_e(0)
    wait_sd(0)
    issue_gather(0)
    for j in range(CPW):
        guarded(j, lambda j=j: (wait_e(j), wait_gather(j)))
        if j + 1 < CPW:
            guarded(j + 1, lambda j=j: (wait_sd(j + 1), issue_gather(j + 1)))
        guarded(j, lambda j=j: compute(j))
        if j + 1 < CPW:
            guarded(j + 1, lambda j=j: issue_e(j + 1))
        guarded(j, lambda j=j: scatter(j))
        if j + 2 < CPW:
            guarded(j + 2, lambda j=j: issue_sd(j + 2))
    plsc.subcore_barrier()

    for t in range(RPS // RCP):
        base = sid * RPS + t * RCP
        pltpu.sync_copy(acc.at[pl.ds(base, RCP)], rows.at[0, pl.ds(0, RCP)])
        pltpu.sync_copy(rows.at[0, pl.ds(0, RCP)],
                        out_hbm.at[cid, pl.ds(base, RCP)])

    @pl.when(sid == 0)
    def _():
        pltpu.sync_copy(acc.at[pl.ds(16 * RPS, RTAIL)],
                        rows.at[0, pl.ds(0, RTAIL)])
        pltpu.sync_copy(rows.at[0, pl.ds(0, RTAIL)],
                        out_hbm.at[cid, pl.ds(16 * RPS, RTAIL)])


@functools.cache
def _sc_edge_kernel():
    return pl.kernel(
        _sc_edge_body,
        out_type=jax.ShapeDtypeStruct((2, N, H), jnp.float32),
        mesh=plsc.VectorSubcoreMesh(core_axis_name="c",
                                    subcore_axis_name="s"),
        scratch_types=[
            pltpu.VMEM((2, 1, EC), jnp.int32),
            pltpu.VMEM((2, 1, EC), jnp.int32),
            pltpu.VMEM((EC, H), jnp.float32),
            pltpu.VMEM((2, EC, H), jnp.float32),
            pltpu.VMEM_SHARED((N, H), jnp.float32),
            pltpu.SemaphoreType.DMA,
            pltpu.SemaphoreType.DMA,
            pltpu.SemaphoreType.DMA,
            pltpu.SemaphoreType.DMA,
            pltpu.SemaphoreType.DMA,
        ],
    )


def _sc_edge(hl, e, src2d, dst2d):
    return _sc_edge_kernel()(hl, e, src2d, dst2d)


@jax.jit
def _run(x, edge_index, edge_attr, edge_attr_v2, batch, W_init, b_init,
         We1, be1, We2, be2, W_layers, b_layers, gamma, beta,
         Wvn1, bvn1, Wvn2, bvn2, W_out, b_out):
    src2d = edge_index[0].reshape(NCHUNK, 1, EC)
    dst2d = edge_index[1].reshape(NCHUNK, 1, EC)
    batch2d = batch.reshape(N, 1)

    e = _edge_embed(edge_attr, edge_attr_v2, We1, We2, be1 + be2)
    hl = _node_init(x, W_init, b_init)
    vn = jnp.zeros((G, H), jnp.float32)

    for l in range(L - 1):
        agg = _sc_edge(hl, e, src2d, dst2d)
        hl, vn = _fused_layer(agg, hl, batch2d, vn, W_layers[l],
                              b_layers[l], gamma[l], beta[l], Wvn1[l],
                              bvn1[l], Wvn2[l], bvn2[l])

    agg = _sc_edge(hl, e, src2d, dst2d)
    return _final(agg, hl, W_layers[L - 1], b_layers[L - 1],
                  gamma[L - 1], beta[L - 1], W_out, b_out)


def kernel(x, edge_index, edge_attr, edge_attr_v2, batch, W_init, b_init,
           We1, be1, We2, be2, W_layers, b_layers, gamma, beta,
           Wvn1, bvn1, Wvn2, bvn2, W_out, b_out):
    return _run(x, edge_index, edge_attr, edge_attr_v2, batch, W_init,
                b_init, We1, be1, We2, be2, W_layers, b_layers, gamma,
                beta, Wvn1, bvn1, Wvn2, bvn2, W_out, b_out)


# speedup vs baseline: 1.8877x; 1.0021x over previous
import functools

import jax
import jax.numpy as jnp
from jax import lax
from jax.experimental import pallas as pl
from jax.experimental.pallas import tpu as pltpu, tpu_sc as plsc

N = 10000
E = 320000
D = 128
H = 128
DE = 16
L = 4
G = 64

BN = 2000
BE = 8000
EC = 128
NCHUNK = E // EC
NW = 32
CPW = (NCHUNK + NW - 1) // NW
RPS = 624
RCP = 104
RTAIL = N - 16 * RPS


def _init_body(x_ref, w_ref, b_ref, o_ref):
    o_ref[...] = jnp.dot(x_ref[...], w_ref[...],
                         preferred_element_type=jnp.float32) + b_ref[...]


def _node_init(x, w, b):
    return pl.pallas_call(
        _init_body,
        grid=(N // BN,),
        in_specs=[
            pl.BlockSpec((BN, D), lambda i: (i, 0)),
            pl.BlockSpec((D, H), lambda i: (0, 0)),
            pl.BlockSpec((1, H), lambda i: (0, 0)),
        ],
        out_specs=pl.BlockSpec((BN, H), lambda i: (i, 0)),
        out_shape=jax.ShapeDtypeStruct((N, H), jnp.float32),
    )(x, w, b.reshape(1, H))


def _edge_body(a1_ref, a2_ref, w1_ref, w2_ref, b_ref, o_ref):
    o_ref[...] = (jnp.dot(a1_ref[...], w1_ref[...],
                          preferred_element_type=jnp.float32)
                  + jnp.dot(a2_ref[...], w2_ref[...],
                            preferred_element_type=jnp.float32)
                  + b_ref[...])


def _edge_embed(ea, ea2, w1, w2, b12):
    return pl.pallas_call(
        _edge_body,
        grid=(E // BE,),
        in_specs=[
            pl.BlockSpec((BE, DE), lambda i: (i, 0)),
            pl.BlockSpec((BE, DE), lambda i: (i, 0)),
            pl.BlockSpec((DE, H), lambda i: (0, 0)),
            pl.BlockSpec((DE, H), lambda i: (0, 0)),
            pl.BlockSpec((1, H), lambda i: (0, 0)),
        ],
        out_specs=pl.BlockSpec((BE, H), lambda i: (i, 0)),
        out_shape=jax.ShapeDtypeStruct((E, H), jnp.float32),
    )(ea, ea2, w1, w2, b12.reshape(1, H))


def _ln(h, g, b):
    mu = jnp.mean(h, axis=-1, keepdims=True)
    d = h - mu
    var = jnp.mean(d * d, axis=-1, keepdims=True)
    return d * jax.lax.rsqrt(var + 1e-5) * g + b


def _fused_layer_body(agg_ref, hl_ref, bt_ref, vn_ref,
                      w_ref, b_ref, g_ref, be_ref,
                      w1_ref, b1_ref, w2_ref, b2_ref,
                      hlo_ref, vno_ref, hbuf, pool, vns):
    p = pl.program_id(0)
    i = pl.program_id(1)

    @pl.when(p == 0)
    def _():
        hl = hl_ref[...]
        sm = agg_ref[0] + agg_ref[1] + hl
        h = (jnp.dot(sm, w_ref[...], preferred_element_type=jnp.float32)
             + b_ref[...])
        hbuf[pl.ds(i * BN, BN), :] = _ln(h, g_ref[...], be_ref[...])
        onehot = (bt_ref[...] == lax.broadcasted_iota(jnp.int32, (1, G), 1)
                  ).astype(jnp.float32)
        contrib = lax.dot_general(onehot, hl, (((0,), (0,)), ((), ())),
                                  preferred_element_type=jnp.float32)

        @pl.when(i == 0)
        def _():
            pool[...] = contrib

        @pl.when(i != 0)
        def _():
            pool[...] += contrib

    @pl.when(p == 1)
    def _():
        @pl.when(i == 0)
        def _():
            t = pool[...] + vn_ref[...]
            t = jnp.maximum(jnp.dot(t, w1_ref[...],
                                    preferred_element_type=jnp.float32)
                            + b1_ref[...], 0.0)
            t = jnp.maximum(jnp.dot(t, w2_ref[...],
                                    preferred_element_type=jnp.float32)
                            + b2_ref[...], 0.0)
            vns[...] = t
            vno_ref[...] = t

        onehot = (bt_ref[...] == lax.broadcasted_iota(jnp.int32, (1, G), 1)
                  ).astype(jnp.float32)
        hlo_ref[...] = (hbuf[pl.ds(i * BN, BN), :]
                        + jnp.dot(onehot, vns[...],
                                  preferred_element_type=jnp.float32))


def _fused_layer(agg, hl, batch2d, vn, w, b, g, be, w1, b1, w2, b2):
    zero = lambda p, i: (0, 0)
    return pl.pallas_call(
        _fused_layer_body,
        grid=(2, N // BN),
        in_specs=[
            pl.BlockSpec((2, BN, H), lambda p, i: (0, (1 - p) * i, 0)),
            pl.BlockSpec((BN, H), lambda p, i: ((1 - p) * i, 0)),
            pl.BlockSpec((BN, 1), lambda p, i: (i, 0)),
            pl.BlockSpec((G, H), zero),
            pl.BlockSpec((H, H), zero),
            pl.BlockSpec((1, H), zero),
            pl.BlockSpec((1, H), zero),
            pl.BlockSpec((1, H), zero),
            pl.BlockSpec((H, H), zero),
            pl.BlockSpec((1, H), zero),
            pl.BlockSpec((H, H), zero),
            pl.BlockSpec((1, H), zero),
        ],
        out_specs=[
            pl.BlockSpec((BN, H), lambda p, i: (p * i, 0)),
            pl.BlockSpec((G, H), zero),
        ],
        out_shape=[
            jax.ShapeDtypeStruct((N, H), jnp.float32),
            jax.ShapeDtypeStruct((G, H), jnp.float32),
        ],
        scratch_shapes=[
            pltpu.VMEM((N, H), jnp.float32),
            pltpu.VMEM((G, H), jnp.float32),
            pltpu.VMEM((G, H), jnp.float32),
        ],
    )(agg, hl, batch2d, vn, w, b.reshape(1, H), g.reshape(1, H),
      be.reshape(1, H), w1, b1.reshape(1, H), w2, b2.reshape(1, H))


def _final_body(agg_ref, hl_ref, w_ref, b_ref, g_ref, be_ref,
                wo_ref, bo_ref, o_ref):
    s = agg_ref[0] + agg_ref[1] + hl_ref[...]
    h = jnp.dot(s, w_ref[...], preferred_element_type=jnp.float32) + b_ref[...]
    h = _ln(h, g_ref[...], be_ref[...])
    o_ref[...] = jnp.maximum(
        jnp.dot(h, wo_ref[...], preferred_element_type=jnp.float32)
        + bo_ref[...], 0.0)


def _final(agg, hl, w, b, g, be, wo, bo):
    return pl.pallas_call(
        _final_body,
        grid=(N // BN,),
        in_specs=[
            pl.BlockSpec((2, BN, H), lambda i: (0, i, 0)),
            pl.BlockSpec((BN, H), lambda i: (i, 0)),
            pl.BlockSpec((H, H), lambda i: (0, 0)),
            pl.BlockSpec((1, H), lambda i: (0, 0)),
            pl.BlockSpec((1, H), lambda i: (0, 0)),
            pl.BlockSpec((1, H), lambda i: (0, 0)),
            pl.BlockSpec((H, H), lambda i: (0, 0)),
            pl.BlockSpec((1, H), lambda i: (0, 0)),
        ],
        out_specs=pl.BlockSpec((BN, H), lambda i: (i, 0)),
        out_shape=jax.ShapeDtypeStruct((N, H), jnp.float32),
    )(agg, hl, w, b.reshape(1, H), g.reshape(1, H), be.reshape(1, H),
      wo, bo.reshape(1, H))


def _sc_edge_body(h_hbm, e_hbm, src_hbm, dst_hbm, out_hbm,
                  srcv, dstv, ev, rows, acc,
                  sg0, sg1, ss0, ss1, sd0, sd1, sd2, se, sc0, sc1):
    cid = lax.axis_index("c")
    sid = lax.axis_index("s")
    w = sid * 2 + cid
    sem_g = (sg0, sg1)
    sem_src = (ss0, ss1)
    sem_dst = (sd0, sd1, sd2)
    sem_sc = (sc0, sc1)
    LAST = CPW - 1
    NTAILW = NCHUNK - LAST * NW
    w_valid = w < NTAILW

    def guarded(j, fn):
        if j < LAST:
            fn()
        else:
            pl.when(w_valid)(lambda: (fn(), None)[1])

    def issue_src(j):
        b = j & 1
        pltpu.async_copy(src_hbm.at[j * NW + w], srcv.at[b], sem_src[b])

    def wait_src(j):
        b = j & 1
        pltpu.make_async_copy(src_hbm.at[j * NW + w], srcv.at[b],
                              sem_src[b]).wait()

    def issue_dst(j):
        d = j % 3
        pltpu.async_copy(dst_hbm.at[j * NW + w], dstv.at[d], sem_dst[d])

    def wait_dst(j):
        d = j % 3
        pltpu.make_async_copy(dst_hbm.at[j * NW + w], dstv.at[d],
                              sem_dst[d]).wait()

    def issue_e(j):
        pltpu.async_copy(e_hbm.at[pl.ds((j * NW + w) * EC, EC)], ev, se)

    def wait_e(j):
        pltpu.make_async_copy(e_hbm.at[pl.ds((j * NW + w) * EC, EC)], ev,
                              se).wait()

    def issue_gather(j):
        b = j & 1
        pltpu.async_copy(h_hbm.at[srcv.at[b, 0]], rows.at[b], sem_g[b])

    def wait_gather(j):
        b = j & 1
        pltpu.make_async_copy(h_hbm.at[srcv.at[b, 0]], rows.at[b],
                              sem_g[b]).wait()

    def compute(j):
        b = j & 1

        def _row(r, cc):
            for jj in range(8):
                sl = pl.ds(jj * 16, 16)
                v = rows[b, r, sl] + ev[r, sl]
                rows[b, r, sl] = jnp.maximum(v, 0.0)
            return cc

        lax.fori_loop(0, EC, _row, 0)

    def issue_scatter(j):
        b = j & 1
        d = j % 3
        pltpu.async_copy(rows.at[b], acc.at[dstv.at[d, 0]], sem_sc[b],
                         add=True)

    def wait_scatter(j):
        b = j & 1
        d = j % 3
        pltpu.make_async_copy(rows.at[b], acc.at[dstv.at[d, 0]],
                              sem_sc[b]).wait()

    def _zrow(r, c):
        for jj in range(8):
            rows[0, r, pl.ds(jj * 16, 16)] = jnp.zeros((16,), jnp.float32)
        return c

    lax.fori_loop(0, EC, _zrow, 0)
    for t in range(RPS // RCP):
        pltpu.sync_copy(rows.at[0, pl.ds(0, RCP)],
                        acc.at[pl.ds(sid * RPS + t * RCP, RCP)])

    @pl.when(sid == 0)
    def _():
        pltpu.sync_copy(rows.at[0, pl.ds(0, RTAIL)],
                        acc.at[pl.ds(16 * RPS, RTAIL)])

    plsc.subcore_barrier()

    issue_src(0)
    issue_src(1)
    issue_dst(0)
    issue_dst(1)
    issue_e(0)
    wait_src(0)
    issue_gather(0)
    for j in range(CPW):
        guarded(j, lambda j=j: (wait_e(j), wait_gather(j)))
        if j >= 1:
            guarded(j - 1, lambda j=j: wait_scatter(j - 1))
        if j + 2 < CPW:
            guarded(j + 2, lambda j=j: issue_dst(j + 2))
        if j + 1 < CPW:
            guarded(j + 1, lambda j=j: (wait_src(j + 1), issue_gather(j + 1)))
        guarded(j, lambda j=j: compute(j))
        if j + 1 < CPW:
            guarded(j + 1, lambda j=j: issue_e(j + 1))
        guarded(j, lambda j=j: (wait_dst(j), issue_scatter(j)))
        if j + 2 < CPW:
            guarded(j + 2, lambda j=j: issue_src(j + 2))
    guarded(CPW - 1, lambda: wait_scatter(CPW - 1))
    plsc.subcore_barrier()

    for t in range(RPS // RCP):
        base = sid * RPS + t * RCP
        pltpu.sync_copy(acc.at[pl.ds(base, RCP)], rows.at[0, pl.ds(0, RCP)])
        pltpu.sync_copy(rows.at[0, pl.ds(0, RCP)],
                        out_hbm.at[cid, pl.ds(base, RCP)])

    @pl.when(sid == 0)
    def _():
        pltpu.sync_copy(acc.at[pl.ds(16 * RPS, RTAIL)],
                        rows.at[0, pl.ds(0, RTAIL)])
        pltpu.sync_copy(rows.at[0, pl.ds(0, RTAIL)],
                        out_hbm.at[cid, pl.ds(16 * RPS, RTAIL)])


@functools.cache
def _sc_edge_kernel():
    return pl.kernel(
        _sc_edge_body,
        out_type=jax.ShapeDtypeStruct((2, N, H), jnp.float32),
        mesh=plsc.VectorSubcoreMesh(core_axis_name="c",
                                    subcore_axis_name="s"),
        scratch_types=[
            pltpu.VMEM((2, 1, EC), jnp.int32),
            pltpu.VMEM((3, 1, EC), jnp.int32),
            pltpu.VMEM((EC, H), jnp.float32),
            pltpu.VMEM((2, EC, H), jnp.float32),
            pltpu.VMEM_SHARED((N, H), jnp.float32),
            pltpu.SemaphoreType.DMA,
            pltpu.SemaphoreType.DMA,
            pltpu.SemaphoreType.DMA,
            pltpu.SemaphoreType.DMA,
            pltpu.SemaphoreType.DMA,
            pltpu.SemaphoreType.DMA,
            pltpu.SemaphoreType.DMA,
            pltpu.SemaphoreType.DMA,
            pltpu.SemaphoreType.DMA,
            pltpu.SemaphoreType.DMA,
        ],
    )


def _sc_edge(hl, e, src2d, dst2d):
    return _sc_edge_kernel()(hl, e, src2d, dst2d)


@jax.jit
def _run(x, edge_index, edge_attr, edge_attr_v2, batch, W_init, b_init,
         We1, be1, We2, be2, W_layers, b_layers, gamma, beta,
         Wvn1, bvn1, Wvn2, bvn2, W_out, b_out):
    src2d = edge_index[0].reshape(NCHUNK, 1, EC)
    dst2d = edge_index[1].reshape(NCHUNK, 1, EC)
    batch2d = batch.reshape(N, 1)

    e = _edge_embed(edge_attr, edge_attr_v2, We1, We2, be1 + be2)
    hl = _node_init(x, W_init, b_init)
    vn = jnp.zeros((G, H), jnp.float32)

    for l in range(L - 1):
        agg = _sc_edge(hl, e, src2d, dst2d)
        hl, vn = _fused_layer(agg, hl, batch2d, vn, W_layers[l],
                              b_layers[l], gamma[l], beta[l], Wvn1[l],
                              bvn1[l], Wvn2[l], bvn2[l])

    agg = _sc_edge(hl, e, src2d, dst2d)
    return _final(agg, hl, W_layers[L - 1], b_layers[L - 1],
                  gamma[L - 1], beta[L - 1], W_out, b_out)


def kernel(x, edge_index, edge_attr, edge_attr_v2, batch, W_init, b_init,
           We1, be1, We2, be2, W_layers, b_layers, gamma, beta,
           Wvn1, bvn1, Wvn2, bvn2, W_out, b_out):
    return _run(x, edge_index, edge_attr, edge_attr_v2, batch, W_init,
                b_init, We1, be1, We2, be2, W_layers, b_layers, gamma,
                beta, Wvn1, bvn1, Wvn2, bvn2, W_out, b_out)
